# baseline (speedup 1.0000x reference)
import functools

import jax
import jax.numpy as jnp
from jax.experimental import pallas as pl
from jax.experimental.pallas import tpu as pltpu

_BN = 2048
_BQ = 512
_LOG_2PI = 1.8378770664093453


def _kde_block(n_real, q_ref, d_ref, out_ref, acc_ref):
    k = pl.program_id(1)
    nk = pl.num_programs(1)

    @pl.when(k == 0)
    def _init():
        acc_ref[...] = jnp.zeros_like(acc_ref)

    q = q_ref[...]
    dblk = d_ref[...]

    qx = jax.lax.dot_general(
        q, dblk, (((1,), (1,)), ((), ())),
        preferred_element_type=jnp.float32)
    qa = -0.5 * jnp.sum(q * q, axis=1, keepdims=True)
    xb = -0.5 * jnp.sum(dblk * dblk, axis=1)
    t = jnp.minimum(qx + qa + xb[None, :], 0.0)

    col = k * _BN + jax.lax.broadcasted_iota(jnp.int32, (1, _BN), 1)
    e = jnp.where(col < n_real, jnp.exp(t), 0.0)

    acc_ref[...] += jnp.sum(e, axis=1, keepdims=True)

    @pl.when(k == nk - 1)
    def _finish():
        d_dim = jnp.float32(q_ref.shape[1])
        const = -0.5 * d_dim * _LOG_2PI - jnp.log(jnp.float32(n_real))
        out_ref[...] = jnp.log(acc_ref[...]) + const


@jax.jit
def kernel(queries, data):
    nq, d = queries.shape
    n, _ = data.shape
    npad = pl.cdiv(n, _BN) * _BN
    data_p = jnp.pad(data, ((0, npad - n), (0, 0)))

    grid = (nq // _BQ, npad // _BN)
    out = pl.pallas_call(
        functools.partial(_kde_block, n),
        grid=grid,
        in_specs=[
            pl.BlockSpec((_BQ, d), lambda i, k: (i, 0)),
            pl.BlockSpec((_BN, d), lambda i, k: (k, 0)),
        ],
        out_specs=pl.BlockSpec((_BQ, 1), lambda i, k: (i, 0)),
        out_shape=jax.ShapeDtypeStruct((nq, 1), jnp.float32),
        scratch_shapes=[pltpu.VMEM((_BQ, 1), jnp.float32)],
        compiler_params=pltpu.CompilerParams(
            dimension_semantics=("parallel", "arbitrary")),
    )(queries, data_p)
    return out[:, 0]


# speedup vs baseline: 1.4247x; 1.4247x over previous
import jax
import jax.numpy as jnp
from jax.experimental import pallas as pl
from jax.experimental.pallas import tpu as pltpu

_BN = 2048
_BQ = 512
_LOG_2PI = 1.8378770664093453


def _kde_block(q_ref, d_ref, qa_ref, out_ref, acc_ref):
    k = pl.program_id(1)
    nk = pl.num_programs(1)

    @pl.when(k == 0)
    def _init():
        acc_ref[...] = jnp.zeros_like(acc_ref)

    t = jax.lax.dot_general(
        q_ref[...], d_ref[...], (((1,), (1,)), ((), ())),
        preferred_element_type=jnp.float32)
    e = jnp.exp(t)
    acc_ref[...] += jnp.sum(e, axis=1, keepdims=True)

    @pl.when(k == nk - 1)
    def _finish():
        out_ref[...] = jnp.log(acc_ref[...]) + qa_ref[...]


@jax.jit
def kernel(queries, data):
    nq, d = queries.shape
    n, _ = data.shape
    npad = pl.cdiv(n, _BN) * _BN

    xb = -0.5 * jnp.sum(data * data, axis=1, keepdims=True)
    data_aug = jnp.concatenate([data, xb], axis=1)
    pad_row = jnp.zeros((npad - n, d + 1), jnp.float32).at[:, d].set(-1e30)
    data_aug = jnp.concatenate([data_aug, pad_row], axis=0)
    q_aug = jnp.concatenate(
        [queries, jnp.ones((nq, 1), jnp.float32)], axis=1)
    qa = (-0.5 * jnp.sum(queries * queries, axis=1, keepdims=True)
          - 0.5 * d * _LOG_2PI - jnp.log(jnp.float32(n)))

    grid = (nq // _BQ, npad // _BN)
    out = pl.pallas_call(
        _kde_block,
        grid=grid,
        in_specs=[
            pl.BlockSpec((_BQ, d + 1), lambda i, k: (i, 0)),
            pl.BlockSpec((_BN, d + 1), lambda i, k: (k, 0)),
            pl.BlockSpec((_BQ, 1), lambda i, k: (i, 0)),
        ],
        out_specs=pl.BlockSpec((_BQ, 1), lambda i, k: (i, 0)),
        out_shape=jax.ShapeDtypeStruct((nq, 1), jnp.float32),
        scratch_shapes=[pltpu.VMEM((_BQ, 1), jnp.float32)],
        compiler_params=pltpu.CompilerParams(
            dimension_semantics=("parallel", "arbitrary")),
    )(q_aug, data_aug, qa)
    return out[:, 0]


# speedup vs baseline: 1.8329x; 1.2865x over previous
import jax
import jax.numpy as jnp
from jax.experimental import pallas as pl
from jax.experimental.pallas import tpu as pltpu

_BN = 2048
_BQ = 1024
_LOG_2PI = 1.8378770664093453


def _kde_block(q_ref, d_ref, qa_ref, out_ref, acc_ref):
    k = pl.program_id(1)
    nk = pl.num_programs(1)

    @pl.when(k == 0)
    def _init():
        acc_ref[...] = jnp.zeros_like(acc_ref)

    t = jax.lax.dot_general(
        q_ref[...], d_ref[...], (((1,), (1,)), ((), ())),
        preferred_element_type=jnp.float32)
    e = jnp.exp(t)
    acc_ref[...] += jnp.sum(e, axis=1, keepdims=True)

    @pl.when(k == nk - 1)
    def _finish():
        out_ref[...] = jnp.log(acc_ref[...]) + qa_ref[...]


@jax.jit
def kernel(queries, data):
    nq, d = queries.shape
    n, _ = data.shape
    npad = pl.cdiv(n, _BN) * _BN

    xb = -0.5 * jnp.sum(data * data, axis=1, keepdims=True)
    data_aug = jnp.concatenate([data, xb], axis=1)
    pad_row = jnp.zeros((npad - n, d + 1), jnp.float32).at[:, d].set(-1e30)
    data_aug = jnp.concatenate([data_aug, pad_row], axis=0)
    q_aug = jnp.concatenate(
        [queries, jnp.ones((nq, 1), jnp.float32)], axis=1)
    qa = (-0.5 * jnp.sum(queries * queries, axis=1, keepdims=True)
          - 0.5 * d * _LOG_2PI - jnp.log(jnp.float32(n)))

    grid = (nq // _BQ, npad // _BN)
    out = pl.pallas_call(
        _kde_block,
        grid=grid,
        in_specs=[
            pl.BlockSpec((_BQ, d + 1), lambda i, k: (i, 0)),
            pl.BlockSpec((_BN, d + 1), lambda i, k: (k, 0)),
            pl.BlockSpec((_BQ, 1), lambda i, k: (i, 0)),
        ],
        out_specs=pl.BlockSpec((_BQ, 1), lambda i, k: (i, 0)),
        out_shape=jax.ShapeDtypeStruct((nq, 1), jnp.float32),
        scratch_shapes=[pltpu.VMEM((_BQ, 1), jnp.float32)],
        compiler_params=pltpu.CompilerParams(
            dimension_semantics=("parallel", "arbitrary")),
    )(q_aug, data_aug, qa)
    return out[:, 0]


# speedup vs baseline: 1.9252x; 1.0504x over previous
import jax
import jax.numpy as jnp
from jax.experimental import pallas as pl
from jax.experimental.pallas import tpu as pltpu

_BN = 4096
_BQ = 1024
_LOG_2PI = 1.8378770664093453


def _kde_block(q_ref, d_ref, qa_ref, out_ref, acc_ref):
    k = pl.program_id(1)
    nk = pl.num_programs(1)

    @pl.when(k == 0)
    def _init():
        acc_ref[...] = jnp.zeros_like(acc_ref)

    t = jax.lax.dot_general(
        q_ref[...], d_ref[...], (((1,), (1,)), ((), ())),
        preferred_element_type=jnp.float32)
    e = jnp.exp(t)
    acc_ref[...] += jnp.sum(e, axis=1, keepdims=True)

    @pl.when(k == nk - 1)
    def _finish():
        out_ref[...] = jnp.log(acc_ref[...]) + qa_ref[...]


@jax.jit
def kernel(queries, data):
    nq, d = queries.shape
    n, _ = data.shape
    npad = pl.cdiv(n, _BN) * _BN

    xb = -0.5 * jnp.sum(data * data, axis=1, keepdims=True)
    data_aug = jnp.concatenate([data, xb], axis=1)
    pad_row = jnp.zeros((npad - n, d + 1), jnp.float32).at[:, d].set(-1e30)
    data_aug = jnp.concatenate([data_aug, pad_row], axis=0)
    q_aug = jnp.concatenate(
        [queries, jnp.ones((nq, 1), jnp.float32)], axis=1)
    qa = (-0.5 * jnp.sum(queries * queries, axis=1, keepdims=True)
          - 0.5 * d * _LOG_2PI - jnp.log(jnp.float32(n)))

    grid = (nq // _BQ, npad // _BN)
    out = pl.pallas_call(
        _kde_block,
        grid=grid,
        in_specs=[
            pl.BlockSpec((_BQ, d + 1), lambda i, k: (i, 0)),
            pl.BlockSpec((_BN, d + 1), lambda i, k: (k, 0)),
            pl.BlockSpec((_BQ, 1), lambda i, k: (i, 0)),
        ],
        out_specs=pl.BlockSpec((_BQ, 1), lambda i, k: (i, 0)),
        out_shape=jax.ShapeDtypeStruct((nq, 1), jnp.float32),
        scratch_shapes=[pltpu.VMEM((_BQ, 1), jnp.float32)],
        compiler_params=pltpu.CompilerParams(
            dimension_semantics=("parallel", "arbitrary")),
    )(q_aug, data_aug, qa)
    return out[:, 0]


# speedup vs baseline: 2.2720x; 1.1801x over previous
import jax
import jax.numpy as jnp
from jax.experimental import pallas as pl
from jax.experimental.pallas import tpu as pltpu

_BN = 4096
_BQ = 1024
_LOG_2PI = 1.8378770664093453
_LOG2_E = 1.4426950408889634


def _kde_block(q_ref, d_ref, qa_ref, out_ref, acc_ref):
    k = pl.program_id(1)
    nk = pl.num_programs(1)

    @pl.when(k == 0)
    def _init():
        acc_ref[...] = jnp.zeros_like(acc_ref)

    t = jax.lax.dot_general(
        q_ref[...], d_ref[...], (((1,), (1,)), ((), ())),
        preferred_element_type=jnp.float32)
    e = jnp.exp2(t)
    acc_ref[...] += jnp.sum(e, axis=1, keepdims=True)

    @pl.when(k == nk - 1)
    def _finish():
        out_ref[...] = jnp.log(acc_ref[...]) + qa_ref[...]


@jax.jit
def kernel(queries, data):
    nq, d = queries.shape
    n, _ = data.shape
    npad = pl.cdiv(n, _BN) * _BN

    xb = -0.5 * jnp.sum(data * data, axis=1, keepdims=True)
    data_aug = jnp.concatenate([data, xb], axis=1)
    pad_row = jnp.zeros((npad - n, d + 1), jnp.float32).at[:, d].set(-1e30)
    data_aug = jnp.concatenate([data_aug, pad_row], axis=0).astype(jnp.bfloat16)
    q_aug = (_LOG2_E * jnp.concatenate(
        [queries, jnp.ones((nq, 1), jnp.float32)], axis=1)).astype(jnp.bfloat16)
    qa = (-0.5 * jnp.sum(queries * queries, axis=1, keepdims=True)
          - 0.5 * d * _LOG_2PI - jnp.log(jnp.float32(n)))

    grid = (nq // _BQ, npad // _BN)
    out = pl.pallas_call(
        _kde_block,
        grid=grid,
        in_specs=[
            pl.BlockSpec((_BQ, d + 1), lambda i, k: (i, 0)),
            pl.BlockSpec((_BN, d + 1), lambda i, k: (k, 0)),
            pl.BlockSpec((_BQ, 1), lambda i, k: (i, 0)),
        ],
        out_specs=pl.BlockSpec((_BQ, 1), lambda i, k: (i, 0)),
        out_shape=jax.ShapeDtypeStruct((nq, 1), jnp.float32),
        scratch_shapes=[pltpu.VMEM((_BQ, 1), jnp.float32)],
        compiler_params=pltpu.CompilerParams(
            dimension_semantics=("parallel", "arbitrary")),
    )(q_aug, data_aug, qa)
    return out[:, 0]


# speedup vs baseline: 2.3072x; 1.0155x over previous
import jax
import jax.numpy as jnp
from jax.experimental import pallas as pl
from jax.experimental.pallas import tpu as pltpu

_BN = 5120
_BQ = 1024
_LOG_2PI = 1.8378770664093453
_LOG2_E = 1.4426950408889634


def _kde_block(q_ref, d_ref, qa_ref, out_ref, acc_ref):
    k = pl.program_id(1)
    nk = pl.num_programs(1)

    @pl.when(k == 0)
    def _init():
        acc_ref[...] = jnp.zeros_like(acc_ref)

    t = jax.lax.dot_general(
        q_ref[...], d_ref[...], (((1,), (1,)), ((), ())),
        preferred_element_type=jnp.float32)
    e = jnp.exp2(t)
    acc_ref[...] += jnp.sum(e, axis=1, keepdims=True)

    @pl.when(k == nk - 1)
    def _finish():
        out_ref[...] = jnp.log(acc_ref[...]) + qa_ref[...]


@jax.jit
def kernel(queries, data):
    nq, d = queries.shape
    n, _ = data.shape
    npad = pl.cdiv(n, _BN) * _BN

    xb = -0.5 * jnp.sum(data * data, axis=1, keepdims=True)
    data_aug = jnp.concatenate([data, xb], axis=1)
    pad_row = jnp.zeros((npad - n, d + 1), jnp.float32).at[:, d].set(-1e30)
    data_aug = jnp.concatenate([data_aug, pad_row], axis=0).astype(jnp.bfloat16)
    q_aug = (_LOG2_E * jnp.concatenate(
        [queries, jnp.ones((nq, 1), jnp.float32)], axis=1)).astype(jnp.bfloat16)
    qa = (-0.5 * jnp.sum(queries * queries, axis=1, keepdims=True)
          - 0.5 * d * _LOG_2PI - jnp.log(jnp.float32(n)))

    grid = (nq // _BQ, npad // _BN)
    out = pl.pallas_call(
        _kde_block,
        grid=grid,
        in_specs=[
            pl.BlockSpec((_BQ, d + 1), lambda i, k: (i, 0)),
            pl.BlockSpec((_BN, d + 1), lambda i, k: (k, 0)),
            pl.BlockSpec((_BQ, 1), lambda i, k: (i, 0)),
        ],
        out_specs=pl.BlockSpec((_BQ, 1), lambda i, k: (i, 0)),
        out_shape=jax.ShapeDtypeStruct((nq, 1), jnp.float32),
        scratch_shapes=[pltpu.VMEM((_BQ, 1), jnp.float32)],
        compiler_params=pltpu.CompilerParams(
            dimension_semantics=("parallel", "arbitrary")),
    )(q_aug, data_aug, qa)
    return out[:, 0]
